# Initial kernel scaffold; baseline (speedup 1.0000x reference)
import jax
import jax.numpy as jnp
from jax.experimental import pallas as pl


def kernel(x, edge_index, batch, mask, ids, W_gnn, b_gnn, W1, b1, W2, b2):
    raise NotImplementedError("write your pallas kernel here")


# speedup vs baseline: 1.0505x; 1.0505x over previous
import functools

import jax
import jax.numpy as jnp
from jax import lax
from jax.experimental import pallas as pl
from jax.experimental.pallas import tpu as pltpu

N_NODES = 10000
N_EDGES = 320000
HID = 128
N_GRAPHS = 64

BLK = 2000
NBLK = N_NODES // BLK


def _tc_body(batch_ref, agg_ref, Wg_ref, bg_ref, W1_ref, b1_ref, W2_ref,
             b2_ref, out_ref, sums_ref, counts_ref):
    i = pl.program_id(0)

    @pl.when(i == 0)
    def _init():
        sums_ref[...] = jnp.zeros_like(sums_ref)
        counts_ref[...] = jnp.zeros_like(counts_ref)

    agg = agg_ref[...]
    h = jnp.maximum(
        jnp.dot(agg, Wg_ref[...], preferred_element_type=jnp.float32)
        + bg_ref[...], 0.0)
    b = batch_ref[0]
    gids = lax.broadcasted_iota(jnp.int32, (N_GRAPHS, BLK), 0).astype(
        jnp.float32)
    onehot = (b == gids).astype(jnp.float32)
    sums_ref[...] += jnp.dot(onehot, h, preferred_element_type=jnp.float32)
    counts_ref[...] += jnp.sum(onehot, axis=1, keepdims=True)

    @pl.when(i == NBLK - 1)
    def _final():
        pool = sums_ref[...] / jnp.maximum(counts_ref[...], 1.0)
        t = jnp.dot(pool, W1_ref[...], preferred_element_type=jnp.float32) \
            + b1_ref[...]
        t = jnp.where(t > 0, t, jnp.exp(jnp.minimum(t, 0.0)) - 1.0)
        out_ref[...] = jnp.dot(t, W2_ref[...],
                               preferred_element_type=jnp.float32) + b2_ref[...]


@functools.partial(jax.jit)
def _tc_phase(agg, batch_f, W_gnn, b_gnn, W1, b1, W2, b2):
    batch3 = batch_f.reshape(NBLK, 1, BLK)
    return pl.pallas_call(
        _tc_body,
        grid=(NBLK,),
        in_specs=[
            pl.BlockSpec((1, 1, BLK), lambda i: (i, 0, 0)),
            pl.BlockSpec((BLK, HID), lambda i: (i, 0)),
            pl.BlockSpec((HID, HID), lambda i: (0, 0)),
            pl.BlockSpec((1, HID), lambda i: (0, 0)),
            pl.BlockSpec((HID, HID), lambda i: (0, 0)),
            pl.BlockSpec((1, HID), lambda i: (0, 0)),
            pl.BlockSpec((HID, 2), lambda i: (0, 0)),
            pl.BlockSpec((1, 2), lambda i: (0, 0)),
        ],
        out_specs=pl.BlockSpec((N_GRAPHS, 2), lambda i: (0, 0)),
        out_shape=jax.ShapeDtypeStruct((N_GRAPHS, 2), jnp.float32),
        scratch_shapes=[
            pltpu.VMEM((N_GRAPHS, HID), jnp.float32),
            pltpu.VMEM((N_GRAPHS, HID), jnp.float32),
        ],
    )(batch3, agg, W_gnn, b_gnn.reshape(1, HID), W1, b1.reshape(1, HID),
      W2, b2.reshape(1, 2))


def kernel(x, edge_index, batch, mask, ids, W_gnn, b_gnn, W1, b1, W2, b2):
    src = edge_index[0].astype(jnp.int32)
    dst = edge_index[1].astype(jnp.int32)
    msgs = x[src] * mask[:, None]
    agg = jax.ops.segment_sum(msgs, dst, num_segments=N_NODES)
    batch_f = batch.astype(jnp.float32)
    out = _tc_phase(agg, batch_f, W_gnn, b_gnn, W1, b1, W2, b2)
    return jnp.squeeze(out)


# speedup vs baseline: 5.2625x; 5.0093x over previous
import functools

import jax
import jax.numpy as jnp
from jax import lax
from jax.experimental import pallas as pl
from jax.experimental.pallas import tpu as pltpu
from jax.experimental.pallas import tpu_sc as plsc

N_NODES = 10000
N_EDGES = 320000
HID = 128
N_GRAPHS = 64

BLK = 2000
NBLK = N_NODES // BLK

NC, NS = 2, 16
NW = NC * NS
SUB = 128
NSUB = N_EDGES // SUB
FULL = NSUB // NW
EXTRA = NSUB - FULL * NW
WR = 624
ZROWS = 208
VLANE = 16

_sc_mesh = plsc.VectorSubcoreMesh(core_axis_name="c", subcore_axis_name="s")


@functools.partial(
    pl.kernel,
    out_type=jax.ShapeDtypeStruct((NC, N_NODES, HID), jnp.float32),
    mesh=_sc_mesh,
    scratch_types=[
        pltpu.VMEM((SUB,), jnp.int32),
        pltpu.VMEM((SUB,), jnp.int32),
        pltpu.VMEM((SUB,), jnp.float32),
        pltpu.VMEM((SUB, HID), jnp.float32),
        pltpu.VMEM((ZROWS, HID), jnp.float32),
        pltpu.VMEM_SHARED((N_NODES, HID), jnp.float32),
        pltpu.SemaphoreType.DMA,
    ],
)
def _sc_agg(x_hbm, src_hbm, dst_hbm, mask_hbm, out_hbm,
            src_v, dst_v, mask_v, rows_v, zero_v, acc, sem):
    cid = lax.axis_index("c")
    sid = lax.axis_index("s")
    wid = sid * NC + cid
    rbase = sid * WR

    def _zrow(r, _):
        for j in range(HID // VLANE):
            zero_v[r, pl.ds(j * VLANE, VLANE)] = jnp.zeros((VLANE,),
                                                           jnp.float32)
        return 0
    lax.fori_loop(0, ZROWS, _zrow, 0)
    for q in range(WR // ZROWS):
        pltpu.sync_copy(zero_v, acc.at[pl.ds(rbase + q * ZROWS, ZROWS), :])

    @pl.when(sid == NS - 1)
    def _zero_tail():
        pltpu.sync_copy(zero_v.at[pl.ds(0, N_NODES - NS * WR), :],
                        acc.at[pl.ds(NS * WR, N_NODES - NS * WR), :])
    plsc.subcore_barrier()

    nsub = FULL + jnp.where(wid < EXTRA, 1, 0)

    def _chunk(c, _):
        base = (wid + NW * c) * SUB
        pltpu.sync_copy(src_hbm.at[pl.ds(base, SUB)], src_v)
        pltpu.sync_copy(dst_hbm.at[pl.ds(base, SUB)], dst_v)
        pltpu.sync_copy(mask_hbm.at[pl.ds(base, SUB)], mask_v)
        pltpu.async_copy(x_hbm.at[src_v], rows_v, sem).wait()

        def _scale(g, _):
            m16 = mask_v[pl.ds(g * VLANE, VLANE)]
            for k in range(VLANE):
                mv = jnp.full((VLANE,), m16[k], jnp.float32)
                r = g * VLANE + k
                for j in range(HID // VLANE):
                    sl = pl.ds(j * VLANE, VLANE)
                    rows_v[r, sl] = rows_v[r, sl] * mv
            return 0
        lax.fori_loop(0, SUB // VLANE, _scale, 0)
        pltpu.sync_copy(rows_v, acc.at[dst_v], add=True)
        return 0
    lax.fori_loop(0, nsub, _chunk, 0)
    plsc.subcore_barrier()
    pltpu.sync_copy(acc.at[pl.ds(rbase, WR), :],
                    out_hbm.at[cid, pl.ds(rbase, WR), :])

    @pl.when(sid == NS - 1)
    def _write_tail():
        pltpu.sync_copy(acc.at[pl.ds(NS * WR, N_NODES - NS * WR), :],
                        out_hbm.at[cid, pl.ds(NS * WR, N_NODES - NS * WR), :])


def _tc_body(batch_ref, agg0_ref, agg1_ref, Wg_ref, bg_ref, W1_ref, b1_ref,
             W2_ref, b2_ref, out_ref, sums_ref, counts_ref):
    i = pl.program_id(0)

    @pl.when(i == 0)
    def _init():
        sums_ref[...] = jnp.zeros_like(sums_ref)
        counts_ref[...] = jnp.zeros_like(counts_ref)

    agg = agg0_ref[...] + agg1_ref[...]
    h = jnp.maximum(
        jnp.dot(agg, Wg_ref[...], preferred_element_type=jnp.float32)
        + bg_ref[...], 0.0)
    b = batch_ref[0]
    gids = lax.broadcasted_iota(jnp.int32, (N_GRAPHS, BLK), 0).astype(
        jnp.float32)
    onehot = (b == gids).astype(jnp.float32)
    sums_ref[...] += jnp.dot(onehot, h, preferred_element_type=jnp.float32)
    counts_ref[...] += jnp.sum(onehot, axis=1, keepdims=True)

    @pl.when(i == NBLK - 1)
    def _final():
        pool = sums_ref[...] / jnp.maximum(counts_ref[...], 1.0)
        t = jnp.dot(pool, W1_ref[...], preferred_element_type=jnp.float32) \
            + b1_ref[...]
        t = jnp.where(t > 0, t, jnp.exp(jnp.minimum(t, 0.0)) - 1.0)
        out_ref[...] = jnp.dot(t, W2_ref[...],
                               preferred_element_type=jnp.float32) + b2_ref[...]


@functools.partial(jax.jit)
def _tc_phase(agg0, agg1, batch_f, W_gnn, b_gnn, W1, b1, W2, b2):
    batch3 = batch_f.reshape(NBLK, 1, BLK)
    return pl.pallas_call(
        _tc_body,
        grid=(NBLK,),
        in_specs=[
            pl.BlockSpec((1, 1, BLK), lambda i: (i, 0, 0)),
            pl.BlockSpec((BLK, HID), lambda i: (i, 0)),
            pl.BlockSpec((BLK, HID), lambda i: (i, 0)),
            pl.BlockSpec((HID, HID), lambda i: (0, 0)),
            pl.BlockSpec((1, HID), lambda i: (0, 0)),
            pl.BlockSpec((HID, HID), lambda i: (0, 0)),
            pl.BlockSpec((1, HID), lambda i: (0, 0)),
            pl.BlockSpec((HID, 2), lambda i: (0, 0)),
            pl.BlockSpec((1, 2), lambda i: (0, 0)),
        ],
        out_specs=pl.BlockSpec((N_GRAPHS, 2), lambda i: (0, 0)),
        out_shape=jax.ShapeDtypeStruct((N_GRAPHS, 2), jnp.float32),
        scratch_shapes=[
            pltpu.VMEM((N_GRAPHS, HID), jnp.float32),
            pltpu.VMEM((N_GRAPHS, HID), jnp.float32),
        ],
    )(batch3, agg0, agg1, W_gnn, b_gnn.reshape(1, HID), W1,
      b1.reshape(1, HID), W2, b2.reshape(1, 2))


def kernel(x, edge_index, batch, mask, ids, W_gnn, b_gnn, W1, b1, W2, b2):
    src = edge_index[0].astype(jnp.int32)
    dst = edge_index[1].astype(jnp.int32)
    part = _sc_agg(x, src, dst, mask)
    batch_f = batch.astype(jnp.float32)
    out = _tc_phase(part[0], part[1], batch_f, W_gnn, b_gnn, W1, b1, W2, b2)
    return jnp.squeeze(out)
